# baseline (speedup 1.0000x reference)
import functools
import math

import jax
import jax.numpy as jnp
from jax.experimental import pallas as pl


def _rep_kernel(bank_ref, q_ref, phi_ref, *, bm, m, b):
    i = pl.program_id(0)
    rows = bank_ref[...]
    q_ref[...] = jnp.broadcast_to(rows[None], (b,) + rows.shape)
    col = jax.lax.broadcasted_iota(jnp.int32, (1, 1, bm), 2).astype(jnp.float32)
    base = (i * bm).astype(jnp.float32)
    phi_ref[...] = (2.0 * math.pi / m) * (base + col)


def kernel(key_embed, bank):
    b = key_embed.shape[0]
    m, dim = bank.shape
    bm = 4000
    grid = (m // bm,)
    q, phi3d = pl.pallas_call(
        functools.partial(_rep_kernel, bm=bm, m=m, b=b),
        grid=grid,
        in_specs=[pl.BlockSpec((bm, dim), lambda i: (i, 0))],
        out_specs=[
            pl.BlockSpec((b, bm, dim), lambda i: (0, i, 0)),
            pl.BlockSpec((1, 1, bm), lambda i: (i, 0, 0)),
        ],
        out_shape=[
            jax.ShapeDtypeStruct((b, m, dim), jnp.float32),
            jax.ShapeDtypeStruct((m // bm, 1, bm), jnp.float32),
        ],
    )(bank)
    q_valid = jnp.ones((b, m), dtype=bool)
    return (q, q_valid, phi3d.reshape(m))


# speedup vs baseline: 1.0097x; 1.0097x over previous
import functools
import math

import jax
import jax.numpy as jnp
from jax.experimental import pallas as pl
from jax.experimental.pallas import tpu as pltpu


def _rep_kernel(bank_ref, q_ref, phi_ref, *, bm, m, b):
    i = pl.program_id(0)
    rows = bank_ref[...]
    for j in range(b):
        q_ref[j, :, :] = rows
    col = jax.lax.broadcasted_iota(jnp.int32, (1, 1, bm), 2).astype(jnp.float32)
    base = (i * bm).astype(jnp.float32)
    phi_ref[...] = (2.0 * math.pi / m) * (base + col)


def kernel(key_embed, bank):
    b = key_embed.shape[0]
    m, dim = bank.shape
    bm = 10000
    grid = (m // bm,)
    q, phi3d = pl.pallas_call(
        functools.partial(_rep_kernel, bm=bm, m=m, b=b),
        grid=grid,
        in_specs=[pl.BlockSpec((bm, dim), lambda i: (i, 0))],
        out_specs=[
            pl.BlockSpec((b, bm, dim), lambda i: (0, i, 0)),
            pl.BlockSpec((1, 1, bm), lambda i: (i, 0, 0)),
        ],
        out_shape=[
            jax.ShapeDtypeStruct((b, m, dim), jnp.float32),
            jax.ShapeDtypeStruct((m // bm, 1, bm), jnp.float32),
        ],
        compiler_params=pltpu.CompilerParams(
            dimension_semantics=("parallel",),
        ),
    )(bank)
    q_valid = jnp.ones((b, m), dtype=bool)
    return (q, q_valid, phi3d.reshape(m))
